# baseline (speedup 1.0000x reference)
import functools
import math

import jax
import jax.numpy as jnp
from jax import lax
from jax.experimental import pallas as pl
from jax.experimental.pallas import tpu as pltpu
from jax.experimental.pallas import tpu_sc as plsc

VOCAB_SIZE = 1000000
D = 64
SCALE = math.sqrt(D)

NC = 2
NS = 16
NW = NC * NS
LANES = 16


def _emb_body(chunk, n_chunks, x_hbm, lut_hbm, out_hbm, idx_v, rows_v, sem):
  wid = lax.axis_index("s") * NC + lax.axis_index("c")
  base = wid * (chunk * n_chunks)

  def per_chunk(g, _):
    off = base + g * chunk
    pltpu.sync_copy(x_hbm.at[pl.ds(off, chunk)], idx_v)
    pltpu.async_copy(lut_hbm.at[idx_v], rows_v, sem).wait()

    def scale_row(r, _):
      for j in range(D // LANES):
        sl = pl.ds(j * LANES, LANES)
        rows_v[r, sl] = rows_v[r, sl] * SCALE
      return 0

    lax.fori_loop(0, chunk, scale_row, 0)
    pltpu.sync_copy(rows_v, out_hbm.at[pl.ds(off, chunk)])
    return 0

  lax.fori_loop(0, n_chunks, per_chunk, 0)


@jax.jit
def kernel(x, lut):
  B, T = x.shape
  n = B * T
  per_w = n // NW
  chunk = 800
  n_chunks = per_w // chunk

  x_flat = x.reshape(n).astype(jnp.int32)

  mesh = plsc.VectorSubcoreMesh(core_axis_name="c", subcore_axis_name="s")
  body = functools.partial(_emb_body, chunk, n_chunks)
  out = pl.kernel(
      body,
      out_type=jax.ShapeDtypeStruct((n, D), jnp.float32),
      mesh=mesh,
      scratch_types=[
          pltpu.VMEM((chunk,), jnp.int32),
          pltpu.VMEM((chunk, D), jnp.float32),
          pltpu.SemaphoreType.DMA,
      ],
      compiler_params=pltpu.CompilerParams(use_tc_tiling_on_sc=False),
  )(x_flat, lut)
  return out.reshape(B, T, D)


# speedup vs baseline: 1.1205x; 1.1205x over previous
import functools
import math

import jax
import jax.numpy as jnp
from jax import lax
from jax.experimental import pallas as pl
from jax.experimental.pallas import tpu as pltpu
from jax.experimental.pallas import tpu_sc as plsc

VOCAB_SIZE = 1000000
D = 64
SCALE = math.sqrt(D)

NC = 2
NS = 16
NW = NC * NS
LANES = 16


def _emb_body(C, n_chunks, per_w, x_hbm, lut_hbm, out_hbm,
              idx_v, rows_g, rows_s, gsem, ssem):
  wid = lax.axis_index("s") * NC + lax.axis_index("c")
  base = wid * per_w
  pltpu.sync_copy(x_hbm.at[pl.ds(base, per_w)], idx_v)

  def start_gather(g, b):
    pltpu.make_async_copy(
        lut_hbm.at[idx_v.at[pl.ds(g * C, C)]], rows_g.at[b], gsem.at[b]
    ).start()

  def wait_gather(b):
    pltpu.make_async_copy(
        lut_hbm.at[idx_v.at[pl.ds(0, C)]], rows_g.at[b], gsem.at[b]
    ).wait()

  def start_store(g, b):
    pltpu.make_async_copy(
        rows_s.at[b], out_hbm.at[pl.ds(base + g * C, C)], ssem.at[b]
    ).start()

  def wait_store(b):
    pltpu.make_async_copy(
        rows_s.at[b], out_hbm.at[pl.ds(base, C)], ssem.at[b]
    ).wait()

  def scale(b):
    @plsc.parallel_loop(0, C, unroll=8)
    def _(r):
      for j in range(D // LANES):
        sl = pl.ds(j * LANES, LANES)
        rows_s[b, r, sl] = rows_g[b, r, sl] * SCALE

  def step(g, b, first, last):
    wait_gather(b)
    if not first:
      wait_store(b)
    scale(b)
    start_store(g, b)
    if not last:
      start_gather(g + 2, b)

  n_pairs = n_chunks // 2
  start_gather(0, 0)
  start_gather(1, 1)
  step(0, 0, True, False)
  step(1, 1, True, False)

  def loop_body(gg, _):
    g0 = gg * 2
    step(g0, 0, False, False)
    step(g0 + 1, 1, False, False)
    return 0

  lax.fori_loop(1, n_pairs - 1, loop_body, 0)
  g0 = (n_pairs - 1) * 2
  step(g0, 0, False, True)
  step(g0 + 1, 1, False, True)
  wait_store(0)
  wait_store(1)


@jax.jit
def kernel(x, lut):
  B, T = x.shape
  n = B * T
  per_w = n // NW
  C = 320
  n_chunks = per_w // C

  x_flat = x.reshape(n).astype(jnp.int32)

  mesh = plsc.VectorSubcoreMesh(core_axis_name="c", subcore_axis_name="s")
  body = functools.partial(_emb_body, C, n_chunks, per_w)
  out = pl.kernel(
      body,
      out_type=jax.ShapeDtypeStruct((n, D), jnp.float32),
      mesh=mesh,
      scratch_types=[
          pltpu.VMEM((per_w,), jnp.int32),
          pltpu.VMEM((2, C, D), jnp.float32),
          pltpu.VMEM((2, C, D), jnp.float32),
          pltpu.SemaphoreType.DMA((2,)),
          pltpu.SemaphoreType.DMA((2,)),
      ],
      compiler_params=pltpu.CompilerParams(use_tc_tiling_on_sc=False),
  )(x_flat, lut)
  return out.reshape(B, T, D)
